# Initial kernel scaffold; baseline (speedup 1.0000x reference)
import jax
import jax.numpy as jnp
from jax.experimental import pallas as pl


def kernel(x, edge_index_adjacent, edge_index_ray, Wl1a, Wr1a, att1a, b1a, Wl1r, Wr1r, att1r, b1r, Wl2a, Wr2a, att2a, b2a, Wl2r, Wr2r, att2r, b2r):
    raise NotImplementedError("write your pallas kernel here")


# speedup vs baseline: 17.1740x; 17.1740x over previous
import functools

import jax
import jax.numpy as jnp
from jax import lax
from jax.experimental import pallas as pl
from jax.experimental.pallas import tpu as pltpu
from jax.experimental.pallas import tpu_sc as plsc

N = 50000
NP = 51200
E = 800000
EREAL = E + N
EP0 = 851968
IN_DIM = 2
C = 64
F = 128

NCORES = 2
NSUB = 16
NW = NCORES * NSUB

RNG = 512
NBUCK = 101
BPAD = 128

EPB = 909312
B1C = 128
CH1 = EPB // (NW * B1C)
SPAN0 = EP0 // NW

_MESH = plsc.VectorSubcoreMesh(core_axis_name="c", subcore_axis_name="s")
_CP = pltpu.CompilerParams(needs_layout_passes=False)


def _proj2(x, w):
    k = w.shape[1]

    def body(xr, wr, outr):
        xb = xr[...]
        wb = wr[...]
        outr[...] = xb[:, 0:1] * wb[0:1, :] + xb[:, 1:2] * wb[1:2, :]

    return pl.pallas_call(
        body,
        out_shape=jax.ShapeDtypeStruct((NP, k), jnp.float32),
        grid=(NP // 1024,),
        in_specs=[pl.BlockSpec((1024, 2), lambda i: (i, 0)),
                  pl.BlockSpec((2, k), lambda i: (0, 0))],
        out_specs=pl.BlockSpec((1024, k), lambda i: (i, 0)),
    )(x, w)


def _proj64(x, w):
    k = w.shape[1]

    def body(xr, wr, outr):
        outr[...] = jnp.dot(xr[...], wr[...],
                            preferred_element_type=jnp.float32)

    return pl.pallas_call(
        body,
        out_shape=jax.ShapeDtypeStruct((NP, k), jnp.float32),
        grid=(NP // 1024,),
        in_specs=[pl.BlockSpec((1024, 64), lambda i: (i, 0)),
                  pl.BlockSpec((64, k), lambda i: (0, 0))],
        out_specs=pl.BlockSpec((1024, k), lambda i: (i, 0)),
    )(x, w)


@functools.partial(
    pl.kernel,
    out_type=jax.ShapeDtypeStruct((NW, BPAD), jnp.int32),
    mesh=_MESH,
    compiler_params=_CP,
    scratch_types=[
        pltpu.VMEM((B1C,), jnp.int32),
        pltpu.VMEM((BPAD + 16,), jnp.int32),
    ],
)
def _bcount(dst_hbm, cnt_hbm, dbuf, cnt):
    cid = lax.axis_index("c")
    sid = lax.axis_index("s")
    w = sid * NCORES + cid
    lanes = lax.iota(jnp.int32, 16)
    m0 = lanes == 0

    @pl.loop(0, BPAD + 16, step=16)
    def _z(i):
        cnt[pl.ds(i, 16)] = jnp.zeros((16,), jnp.int32)

    @pl.loop(0, SPAN0, step=B1C)
    def _chunk(off):
        cb0 = pl.multiple_of(w * SPAN0 + off, 128)
        pltpu.sync_copy(dst_hbm.at[pl.ds(cb0, B1C)], dbuf)

        @pl.loop(0, B1C, step=16)
        def _group(gb):
            bv = lax.shift_right_logical(dbuf[pl.ds(gb, 16)], 9)
            for i in range(16):
                bfull = jnp.full((16,), bv[i], jnp.int32)
                old = plsc.load_gather(cnt, [bfull])
                plsc.store_scatter(cnt, [bfull], old + 1, mask=m0)

    pltpu.sync_copy(cnt.at[pl.ds(0, BPAD)], cnt_hbm.at[w])


@functools.partial(
    pl.kernel,
    out_type=(jax.ShapeDtypeStruct((EPB,), jnp.int32),
              jax.ShapeDtypeStruct((EPB,), jnp.int32)),
    mesh=_MESH,
    compiler_params=_CP,
    scratch_types=[
        pltpu.VMEM((B1C,), jnp.int32),
        pltpu.VMEM((B1C,), jnp.int32),
        pltpu.VMEM((BPAD + 16,), jnp.int32),
        pltpu.VMEM((BPAD + 16,), jnp.int32),
        pltpu.VMEM((NBUCK * 16 + 16,), jnp.int32),
        pltpu.VMEM((NBUCK * 16 + 16,), jnp.int32),
    ],
)
def _bscatter(src_hbm, dst_hbm, off_hbm, sp_hbm, dp_hbm,
              sbuf, dbuf, cur, offv, stage_s, stage_d):
    cid = lax.axis_index("c")
    sid = lax.axis_index("s")
    w = sid * NCORES + cid
    lanes = lax.iota(jnp.int32, 16)
    m0 = lanes == 0

    @pl.loop(0, BPAD + 16, step=16)
    def _z(i):
        cur[pl.ds(i, 16)] = jnp.zeros((16,), jnp.int32)

    pltpu.sync_copy(off_hbm.at[w], offv.at[pl.ds(0, BPAD)])

    @pl.loop(0, SPAN0, step=B1C)
    def _chunk(off):
        cb0 = pl.multiple_of(w * SPAN0 + off, 128)
        pltpu.sync_copy(src_hbm.at[pl.ds(cb0, B1C)], sbuf)
        pltpu.sync_copy(dst_hbm.at[pl.ds(cb0, B1C)], dbuf)

        @pl.loop(0, B1C, step=16)
        def _group(gb):
            sv = sbuf[pl.ds(gb, 16)]
            dv = dbuf[pl.ds(gb, 16)]
            bv = lax.shift_right_logical(dv, 9)
            for i in range(16):
                b = bv[i]
                bfull = jnp.full((16,), b, jnp.int32)
                pos = plsc.load_gather(cur, [bfull])[0]
                slot = lax.bitwise_and(pos, 15)
                bs16 = pl.multiple_of(b * 16, 16)
                tgt = bfull * 16 + slot
                plsc.store_scatter(stage_s, [tgt],
                                   jnp.full((16,), sv[i], jnp.int32),
                                   mask=m0)
                plsc.store_scatter(stage_d, [tgt],
                                   jnp.full((16,), dv[i], jnp.int32),
                                   mask=m0)

                @pl.when(slot == 15)
                def _flush():
                    base = pl.multiple_of(
                        plsc.load_gather(offv, [bfull])[0] + pos - 15, 16)
                    pltpu.sync_copy(stage_s.at[pl.ds(bs16, 16)],
                                    sp_hbm.at[pl.ds(base, 16)])
                    pltpu.sync_copy(stage_d.at[pl.ds(bs16, 16)],
                                    dp_hbm.at[pl.ds(base, 16)])

                plsc.store_scatter(cur, [bfull],
                                   jnp.full((16,), pos + 1, jnp.int32),
                                   mask=m0)

    @pl.loop(0, NBUCK)
    def _tail(b):
        bfull = jnp.full((16,), b, jnp.int32)
        pos = plsc.load_gather(cur, [bfull])[0]

        @pl.when(lax.bitwise_and(pos, 15) != 0)
        def _flush():
            base = pl.multiple_of(
                plsc.load_gather(offv, [bfull])[0]
                + pos - lax.bitwise_and(pos, 15), 16)
            bs16 = pl.multiple_of(b * 16, 16)
            pltpu.sync_copy(stage_s.at[pl.ds(bs16, 16)],
                            sp_hbm.at[pl.ds(base, 16)])
            pltpu.sync_copy(stage_d.at[pl.ds(bs16, 16)],
                            dp_hbm.at[pl.ds(base, 16)])


@functools.partial(
    pl.kernel,
    out_type=jax.ShapeDtypeStruct((EPB * 2,), jnp.float32),
    mesh=_MESH,
    compiler_params=_CP,
    scratch_types=[
        pltpu.VMEM((B1C,), jnp.int32),
        pltpu.VMEM((B1C,), jnp.int32),
        pltpu.VMEM((B1C, F), jnp.float32),
        pltpu.VMEM((B1C, F), jnp.float32),
        pltpu.VMEM((2 * B1C,), jnp.float32),
        pltpu.VMEM((F,), jnp.float32),
    ],
)
def _phase1(xl_hbm, xr_hbm, src_hbm, dst_hbm, att_hbm,
            ex_hbm,
            sbuf, dbuf, xlv, xrv, exb, attv):
    cid = lax.axis_index("c")
    sid = lax.axis_index("s")
    w = sid * NCORES + cid
    lanes = lax.iota(jnp.int32, 16)
    zeros16 = jnp.zeros((16,), jnp.float32)

    pltpu.sync_copy(att_hbm, attv)
    att_r = [attv[pl.ds(r * 16, 16)] for r in range(8)]

    @pl.loop(0, CH1)
    def _chunk(it):
        cbase = pl.multiple_of((w * CH1 + it) * B1C, 128)
        pltpu.sync_copy(src_hbm.at[pl.ds(cbase, B1C)], sbuf)
        pltpu.sync_copy(dst_hbm.at[pl.ds(cbase, B1C)], dbuf)

        @pl.loop(0, B1C, step=16)
        def _cl(gb):
            sbuf[pl.ds(gb, 16)] = jnp.clip(sbuf[pl.ds(gb, 16)], 0, NP - 1)
            dbuf[pl.ds(gb, 16)] = jnp.clip(dbuf[pl.ds(gb, 16)], 0, NP - 1)

        pltpu.sync_copy(xl_hbm.at[sbuf], xlv)
        pltpu.sync_copy(xr_hbm.at[dbuf], xrv)

        @pl.loop(0, B1C, step=16)
        def _group(gb):
            def edge_body(j, carry):
                l0v, l1v = carry
                e = gb + j
                acc0 = zeros16
                acc1 = zeros16
                for r in range(4):
                    t = xlv[e, pl.ds(r * 16, 16)] + xrv[e, pl.ds(r * 16, 16)]
                    t = jnp.maximum(t, 0.2 * t)
                    acc0 = acc0 + t * att_r[r]
                for r in range(4, 8):
                    t = xlv[e, pl.ds(r * 16, 16)] + xrv[e, pl.ds(r * 16, 16)]
                    t = jnp.maximum(t, 0.2 * t)
                    acc1 = acc1 + t * att_r[r]
                l0 = jnp.sum(acc0)
                l1 = jnp.sum(acc1)
                l0v = jnp.where(lanes == j, l0, l0v)
                l1v = jnp.where(lanes == j, l1, l1v)
                return (l0v, l1v)

            l0v, l1v = lax.fori_loop(0, 16, edge_body, (zeros16, zeros16))
            ex0 = jnp.exp(l0v)
            ex1 = jnp.exp(l1v)
            plsc.store_scatter(exb, [(gb + lanes) * 2], ex0)
            plsc.store_scatter(exb, [(gb + lanes) * 2 + 1], ex1)

        pltpu.sync_copy(
            exb, ex_hbm.at[pl.ds(pl.multiple_of(cbase * 2, 256), 2 * B1C)])


@functools.partial(
    pl.kernel,
    out_type=jax.ShapeDtypeStruct((NP, F), jnp.float32),
    mesh=_MESH,
    compiler_params=_CP,
    scratch_types=[
        pltpu.VMEM((B1C,), jnp.int32),
        pltpu.VMEM((B1C,), jnp.int32),
        pltpu.VMEM((2 * B1C,), jnp.float32),
        pltpu.VMEM((B1C, F), jnp.float32),
        pltpu.VMEM((RNG, F), jnp.float32),
        pltpu.VMEM((RNG * 16 + 16,), jnp.float32),
        pltpu.VMEM((32,), jnp.int32),
        pltpu.VMEM((32,), jnp.int32),
    ],
)
def _phase2(xl_hbm, sp_hbm, dp_hbm, ex_hbm, offt_hbm, cntt_hbm,
            out_hbm,
            sbuf, dbuf, exv, rows_v, acc, den, offr, cntr):
    cid = lax.axis_index("c")
    sid = lax.axis_index("s")
    w = sid * NCORES + cid
    lanes = lax.iota(jnp.int32, 16)
    zeros16 = jnp.zeros((16,), jnp.float32)

    @pl.loop(0, 4)
    def _slot(slot):
        b = slot * 32 + w

        @pl.when(b < 100)
        def _dobucket():
            blo = b * RNG

            @pl.loop(0, RNG)
            def _zacc(i):
                for r in range(8):
                    acc[i, pl.ds(r * 16, 16)] = zeros16

            @pl.loop(0, RNG * 16 + 16, step=16)
            def _zden(i):
                den[pl.ds(i, 16)] = zeros16

            pltpu.sync_copy(offt_hbm.at[b], offr)
            pltpu.sync_copy(cntt_hbm.at[b], cntr)

            @pl.loop(0, 32)
            def _run(wp):
                wfull = jnp.full((16,), wp, jnp.int32)
                base = pl.multiple_of(
                    plsc.load_gather(offr, [wfull])[0], 16)
                cnt = plsc.load_gather(cntr, [wfull])[0]
                nch = (cnt + B1C - 1) // B1C

                def chunk_body(ch, _):
                    cb = pl.multiple_of(base + ch * B1C, 16)
                    rem = cnt - ch * B1C
                    pltpu.sync_copy(sp_hbm.at[pl.ds(cb, B1C)], sbuf)
                    pltpu.sync_copy(dp_hbm.at[pl.ds(cb, B1C)], dbuf)
                    cb2 = pl.multiple_of(cb * 2, 32)
                    pltpu.sync_copy(ex_hbm.at[pl.ds(cb2, 2 * B1C)],
                                    exv)

                    @pl.loop(0, B1C, step=16)
                    def _cl(gb):
                        sbuf[pl.ds(gb, 16)] = jnp.clip(
                            sbuf[pl.ds(gb, 16)], 0, NP - 1)

                    pltpu.sync_copy(xl_hbm.at[sbuf], rows_v)

                    @pl.loop(0, B1C, step=16)
                    def _group(gb):
                        @pl.when(gb < rem)
                        def _dogroup():
                            dv = dbuf[pl.ds(gb, 16)]
                            dlv = dv - blo
                            e0v = plsc.load_gather(
                                exv, [(gb + lanes) * 2])
                            e1v = plsc.load_gather(
                                exv, [(gb + lanes) * 2 + 1])
                            for i in range(16):
                                @pl.when(gb + i < rem)
                                def _edge():
                                    dl = dlv[i]
                                    e0 = e0v[i]
                                    e1 = e1v[i]
                                    for r in range(4):
                                        sl = pl.ds(r * 16, 16)
                                        acc[dl, sl] = (
                                            acc[dl, sl]
                                            + rows_v[gb + i, sl] * e0)
                                    for r in range(4, 8):
                                        sl = pl.ds(r * 16, 16)
                                        acc[dl, sl] = (
                                            acc[dl, sl]
                                            + rows_v[gb + i, sl] * e1)
                                    upd = (jnp.where(lanes == 0, e0, 0.0)
                                           + jnp.where(lanes == 8, e1,
                                                       0.0))
                                    dsl = pl.ds(
                                        pl.multiple_of(dl * 16, 16), 16)
                                    den[dsl] = den[dsl] + upd

                    return 0

                lax.fori_loop(0, nch, chunk_body, 0)

            @pl.loop(0, RNG, step=16)
            def _wb(i):
                rl = i + lanes
                iv0 = 1.0 / (plsc.load_gather(den, [rl * 16]) + 1e-16)
                iv1 = 1.0 / (plsc.load_gather(den, [rl * 16 + 8]) + 1e-16)
                for j in range(16):
                    for r in range(4):
                        sl = pl.ds(r * 16, 16)
                        acc[i + j, sl] = acc[i + j, sl] * iv0[j]
                    for r in range(4, 8):
                        sl = pl.ds(r * 16, 16)
                        acc[i + j, sl] = acc[i + j, sl] * iv1[j]

            @pl.loop(0, RNG, step=128)
            def _out(i):
                pltpu.sync_copy(acc.at[pl.ds(i, 128)],
                                out_hbm.at[pl.ds(
                                    pl.multiple_of(blo + i, 128), 128)])


def _bucket(s, d):
    cnts = _bcount(d)
    run16 = ((cnts + 15) // 16) * 16
    btot = jnp.sum(run16, axis=0)
    boff = jnp.cumsum(btot) - btot
    within = jnp.cumsum(run16, axis=0) - run16
    offs = (boff[None, :] + within).astype(jnp.int32)
    sp, dp = _bscatter(s, d, offs)
    offt = jnp.transpose(offs).astype(jnp.int32)
    cntt = jnp.transpose(cnts).astype(jnp.int32)
    return sp, dp, offt, cntt


def _conv(xl, xr, attf, eb):
    sp, dp, offt, cntt = eb
    ex = _phase1(xl, xr, sp, dp, attf)
    return _phase2(xl, sp, dp, ex, offt, cntt)


def _head_mean(o, bias):
    return (o[:, :C] + o[:, C:]) * 0.5 + bias[None, :]


def kernel(x, edge_index_adjacent, edge_index_ray,
           Wl1a, Wr1a, att1a, b1a, Wl1r, Wr1r, att1r, b1r,
           Wl2a, Wr2a, att2a, b2a, Wl2r, Wr2r, att2r, b2r):
    loop = jnp.arange(N, dtype=jnp.int32)
    npad = EP0 - EREAL
    pad_src = jnp.zeros((npad,), jnp.int32)
    pad_dst = jnp.full((npad,), NP, jnp.int32)

    def mk_edges(ei):
        s = jnp.concatenate([ei[0].astype(jnp.int32), loop, pad_src])
        d = jnp.concatenate([ei[1].astype(jnp.int32), loop, pad_dst])
        return s, d

    sa, da = mk_edges(edge_index_adjacent)
    sr, dr = mk_edges(edge_index_ray)
    eba = _bucket(sa, da)
    ebr = _bucket(sr, dr)

    xp = jnp.pad(x.astype(jnp.float32), ((0, NP - N), (0, 0)))
    w1 = jnp.concatenate([Wl1a, Wr1a, Wl1r, Wr1r], axis=1)
    p1 = _proj2(xp, w1)
    xla, xra = p1[:, 0:F], p1[:, F:2 * F]
    xlr, xrr = p1[:, 2 * F:3 * F], p1[:, 3 * F:]

    o1a = _conv(xla, xra, att1a.reshape(-1), eba)
    o1r = _conv(xlr, xrr, att1r.reshape(-1), ebr)
    h1 = jax.nn.relu((_head_mean(o1a, b1a) + _head_mean(o1r, b1r)) * 0.5)

    w2 = jnp.concatenate([Wl2a, Wr2a, Wl2r, Wr2r], axis=1)
    p2 = _proj64(h1, w2)
    xla2, xra2 = p2[:, 0:F], p2[:, F:2 * F]
    xlr2, xrr2 = p2[:, 2 * F:3 * F], p2[:, 3 * F:]

    o2a = _conv(xla2, xra2, att2a.reshape(-1), eba)
    o2r = _conv(xlr2, xrr2, att2r.reshape(-1), ebr)
    h2 = (_head_mean(o2a, b2a) + _head_mean(o2r, b2r)) * 0.5
    return h2[:N]
